# Initial kernel scaffold; baseline (speedup 1.0000x reference)
import jax
import jax.numpy as jnp
from jax.experimental import pallas as pl


def kernel(net_feat, net_edge_index, net_edge_weights, dag_feat, dag_edge_index, dag_edge_weights, action, net_W, net_b, dag_W, dag_b, A1, b1, A2, b2, F1, fb1, F2, fb2):
    raise NotImplementedError("write your pallas kernel here")


# speedup vs baseline: 7.8747x; 7.8747x over previous
import jax
import jax.numpy as jnp
from jax import lax
from jax.experimental import pallas as pl
from jax.experimental.pallas import tpu as pltpu
from jax.experimental.pallas import tpu_sc as plsc

N = 10000
D = 128
H = 64
NP = 10240
SL = NP // 16
NC, NS, L = 2, 16, 16
K = 128
PE_NET = 162 * NS * K
PE_DAG = 84 * NS * K
DEG2 = (NP // D, D)


def _scdeg_body(col_n, ew_n, col_d, ew_d, z1, deg_n_out, deg_d_out,
                col_v, ew_v, degbuf, deg_sp, sem):
    c = lax.axis_index("c")
    s = lax.axis_index("s")
    sl = pl.ds(s * SL, SL)
    pltpu.sync_copy(z1.at[sl], deg_sp.at[sl])
    plsc.subcore_barrier()

    def _deg(col_hbm, ew_hbm, nb):
        def body(i, _):
            base = (s * nb + i) * K
            pltpu.sync_copy(col_hbm.at[pl.ds(base, K)], col_v)
            pltpu.sync_copy(ew_hbm.at[pl.ds(base, K)], ew_v)
            pltpu.sync_copy(ew_v, deg_sp.at[col_v], add=True)
            return 0
        lax.fori_loop(0, nb, body, 0)

    @pl.when(c == 0)
    def _():
        _deg(col_n, ew_n, PE_NET // (NS * K))

    @pl.when(c == 1)
    def _():
        _deg(col_d, ew_d, PE_DAG // (NS * K))

    plsc.subcore_barrier()
    pltpu.sync_copy(deg_sp.at[sl], degbuf)

    @pl.when(c == 0)
    def _():
        pltpu.sync_copy(degbuf, deg_n_out.at[sl])

    @pl.when(c == 1)
    def _():
        pltpu.sync_copy(degbuf, deg_d_out.at[sl])


def _sc_deg(col_n, ew_n, col_d, ew_d, z1):
    f32 = jnp.float32
    kern = pl.kernel(
        _scdeg_body,
        mesh=plsc.VectorSubcoreMesh(core_axis_name="c", subcore_axis_name="s"),
        compiler_params=pltpu.CompilerParams(needs_layout_passes=False, use_tc_tiling_on_sc=False),
        out_type=[
            jax.ShapeDtypeStruct((NP,), f32),
            jax.ShapeDtypeStruct((NP,), f32),
        ],
        scratch_types=[
            pltpu.VMEM((K,), jnp.int32),
            pltpu.VMEM((K,), f32),
            pltpu.VMEM((SL,), f32),
            pltpu.VMEM_SHARED((NP,), f32),
            pltpu.SemaphoreType.DMA,
        ],
    )
    return kern(col_n, ew_n, col_d, ew_d, z1)


def _mm_body(nf, df, nw, dw, dgn, dgd, on, od, dvn, dvd):
    on[...] = jnp.dot(nf[...], nw[...], preferred_element_type=jnp.float32)
    od[...] = jnp.dot(df[...], dw[...], preferred_element_type=jnp.float32)

    @pl.when(pl.program_id(0) == 0)
    def _():
        dvn[...] = jnp.where(dgn[...] > 0, lax.rsqrt(dgn[...]), 0.0)
        dvd[...] = jnp.where(dgd[...] > 0, lax.rsqrt(dgd[...]), 0.0)


def _t1(net_feat, net_W, dag_feat, dag_W, deg_n, deg_d):
    MB = 400
    cs = lambda shape: pl.BlockSpec(shape, lambda m: tuple(0 for _ in shape))
    return pl.pallas_call(
        _mm_body,
        grid=(N // MB,),
        in_specs=[
            pl.BlockSpec((MB, D), lambda m: (m, 0)),
            pl.BlockSpec((MB, D), lambda m: (m, 0)),
            cs((D, D)), cs((D, D)),
            cs(DEG2), cs(DEG2),
        ],
        out_specs=[
            pl.BlockSpec((MB, D), lambda m: (m, 0)),
            pl.BlockSpec((MB, D), lambda m: (m, 0)),
            cs(DEG2), cs(DEG2),
        ],
        out_shape=[
            jax.ShapeDtypeStruct((N, D), jnp.float32),
            jax.ShapeDtypeStruct((N, D), jnp.float32),
            jax.ShapeDtypeStruct(DEG2, jnp.float32),
            jax.ShapeDtypeStruct(DEG2, jnp.float32),
        ],
    )(net_feat, dag_feat, net_W, dag_W, deg_n.reshape(DEG2),
      deg_d.reshape(DEG2))


def _scmsg_body(row_n, col_n, ew_n, row_d, col_d, ew_d, xwi_n, xwi_d,
                dinv_n, dinv_d, z2, acc_n_out, acc_d_out,
                row_v, col_v, ew_v, idx_v, rows_v, normbuf,
                dinv_full_n, dinv_full_d, acc_sp_n, acc_sp_d, sem):
    c = lax.axis_index("c")
    s = lax.axis_index("s")
    sl = pl.ds(s * SL, SL)
    pltpu.sync_copy(z2.at[sl], acc_sp_n.at[sl])
    pltpu.sync_copy(z2.at[sl], acc_sp_d.at[sl])
    pltpu.sync_copy(dinv_n, dinv_full_n)
    pltpu.sync_copy(dinv_d, dinv_full_d)
    plsc.subcore_barrier()

    def _msg(row_hbm, col_hbm, ew_hbm, xwi_hbm, dinv_full, acc_sp, nb):
        def body(i, _):
            base = (s * nb + i) * K
            pltpu.sync_copy(row_hbm.at[pl.ds(base, K)], row_v)
            pltpu.sync_copy(col_hbm.at[pl.ds(base, K)], col_v)
            pltpu.sync_copy(ew_hbm.at[pl.ds(base, K)], ew_v)
            for j in range(K // L):
                r16 = row_v[pl.ds(j * L, L)]
                idx_v[pl.ds(j * L, L)] = r16 * 2 + c
            pltpu.async_copy(xwi_hbm.at[idx_v], rows_v, sem).wait()
            for j in range(K // L):
                r16 = row_v[pl.ds(j * L, L)]
                ew16 = ew_v[pl.ds(j * L, L)]
                dr = plsc.load_gather(dinv_full, [r16])
                normbuf[...] = ew16 * dr
                for e in range(L):
                    ns = plsc.load_gather(
                        normbuf, [jnp.full((L,), e, jnp.int32)])
                    r = j * L + e
                    for cc in range(H // L):
                        rows_v[r, cc * L:(cc + 1) * L] = (
                            rows_v[r, cc * L:(cc + 1) * L] * ns)
            pltpu.sync_copy(rows_v, acc_sp.at[col_v], add=True)
            return 0
        lax.fori_loop(0, nb, body, 0)

    _msg(row_n, col_n, ew_n, xwi_n, dinv_full_n, acc_sp_n, PE_NET // (NS * K))
    _msg(row_d, col_d, ew_d, xwi_d, dinv_full_d, acc_sp_d, PE_DAG // (NS * K))
    plsc.subcore_barrier()

    pltpu.sync_copy(acc_sp_n.at[sl], acc_n_out.at[c, sl])
    pltpu.sync_copy(acc_sp_d.at[sl], acc_d_out.at[c, sl])


def _sc_msg(row_n, col_n, ew_n, row_d, col_d, ew_d, xwi_n, xwi_d,
            dinv_n, dinv_d, z2):
    f32 = jnp.float32
    kern = pl.kernel(
        _scmsg_body,
        mesh=plsc.VectorSubcoreMesh(core_axis_name="c", subcore_axis_name="s"),
        compiler_params=pltpu.CompilerParams(needs_layout_passes=False, use_tc_tiling_on_sc=False),
        out_type=[
            jax.ShapeDtypeStruct((NC, NP, H), f32),
            jax.ShapeDtypeStruct((NC, NP, H), f32),
        ],
        scratch_types=[
            pltpu.VMEM((K,), jnp.int32),
            pltpu.VMEM((K,), jnp.int32),
            pltpu.VMEM((K,), f32),
            pltpu.VMEM((K,), jnp.int32),
            pltpu.VMEM((K, H), f32),
            pltpu.VMEM((L,), f32),
            pltpu.VMEM((NP,), f32),
            pltpu.VMEM((NP,), f32),
            pltpu.VMEM_SHARED((NP, H), f32),
            pltpu.VMEM_SHARED((NP, H), f32),
            pltpu.SemaphoreType.DMA,
        ],
    )
    return kern(row_n, col_n, ew_n, row_d, col_d, ew_d, xwi_n, xwi_d,
                dinv_n, dinv_d, z2)


def _t2_body(acc_n, acc_d, dinv_n, dinv_d, bn, bd, act, A1, b1, A2, b2,
             F1, fb1, F2, fb2, out, s_ref):
    m = pl.program_id(0)
    nblk = pl.num_programs(0)
    BLK = acc_n.shape[1]

    @pl.when(m == 0)
    def _():
        s_ref[...] = jnp.zeros_like(s_ref)

    node = m * BLK + lax.broadcasted_iota(jnp.int32, (BLK, 1), 0)
    mask = (node < N).astype(jnp.float32)

    def half_sum(acc, dinv, b, h):
        v = jax.nn.relu(dinv * acc[h] + b[0:1, h * H:(h + 1) * H])
        return jnp.sum(v * mask, axis=0, keepdims=True)

    s_ref[0:1, 0:H] += half_sum(acc_n, dinv_n[...], bn, 0)
    s_ref[0:1, H:D] += half_sum(acc_n, dinv_n[...], bn, 1)
    s_ref[1:2, 0:H] += half_sum(acc_d, dinv_d[...], bd, 0)
    s_ref[1:2, H:D] += half_sum(acc_d, dinv_d[...], bd, 1)

    @pl.when(m == nblk - 1)
    def _():
        inv_n = jnp.float32(1.0 / N)
        emb_n = s_ref[0:1, :] * inv_n
        emb_d = s_ref[1:2, :] * inv_n
        hh = act[...] @ A1[...] + b1[...]
        hh = hh * jnp.tanh(jax.nn.softplus(hh))
        ae = hh @ A2[...] + b2[...]
        h2 = jax.nn.relu(
            emb_n @ F1[0:D, :] + emb_d @ F1[D:2 * D, :]
            + ae @ F1[2 * D:3 * D, :] + fb1[...])
        sv = jnp.sum(h2 * F2[...].T, axis=1, keepdims=True) + fb2[...]
        out[...] = jnp.broadcast_to(sv, out.shape)


def _t2(acc_n, acc_d, dinv_n, dinv_d, bn, bd, act, A1, b1, A2, b2,
        F1, fb1, F2, fb2):
    BLK = 512
    cs = lambda shape: pl.BlockSpec(shape, lambda m: tuple(0 for _ in shape))
    return pl.pallas_call(
        _t2_body,
        grid=(NP // BLK,),
        in_specs=[
            pl.BlockSpec((NC, BLK, H), lambda m: (0, m, 0)),
            pl.BlockSpec((NC, BLK, H), lambda m: (0, m, 0)),
            pl.BlockSpec((BLK, 1), lambda m: (m, 0)),
            pl.BlockSpec((BLK, 1), lambda m: (m, 0)),
            cs((1, D)), cs((1, D)),
            cs((1, 512)),
            cs((512, D)), cs((1, D)),
            cs((D, D)), cs((1, D)),
            cs((3 * D, D)), cs((1, D)),
            cs((D, 1)), cs((1, 1)),
        ],
        out_specs=pl.BlockSpec((1, D), lambda m: (0, 0)),
        out_shape=jax.ShapeDtypeStruct((1, D), jnp.float32),
        scratch_shapes=[pltpu.VMEM((8, D), jnp.float32)],
    )(acc_n, acc_d, dinv_n, dinv_d, bn, bd, act, A1, b1, A2, b2,
      F1, fb1, F2, fb2)


def _prep(ei, ew, pe):
    e = ew.shape[0]
    loop = jnp.arange(N, dtype=ei.dtype)
    npad = pe - e - N
    row = jnp.concatenate([ei[0], loop, jnp.zeros((npad,), ei.dtype)])
    col = jnp.concatenate([ei[1], loop, jnp.full((npad,), N, ei.dtype)])
    eww = jnp.concatenate(
        [ew, jnp.ones((N,), ew.dtype), jnp.zeros((npad,), ew.dtype)])
    return row, col, eww


def kernel(net_feat, net_edge_index, net_edge_weights, dag_feat,
           dag_edge_index, dag_edge_weights, action, net_W, net_b, dag_W,
           dag_b, A1, b1, A2, b2, F1, fb1, F2, fb2):
    row_n, col_n, ew_n = _prep(net_edge_index, net_edge_weights, PE_NET)
    row_d, col_d, ew_d = _prep(dag_edge_index, dag_edge_weights, PE_DAG)
    z1 = jnp.zeros((NP,), jnp.float32)
    z2 = jnp.zeros((NP, H), jnp.float32)
    deg_n, deg_d = _sc_deg(col_n, ew_n, col_d, ew_d, z1)
    xw_net, xw_dag, dinv_n, dinv_d = _t1(
        net_feat, net_W, dag_feat, dag_W, deg_n, deg_d)
    acc_n, acc_d = _sc_msg(
        row_n, col_n, ew_n, row_d, col_d, ew_d,
        xw_net.reshape(2 * N, H), xw_dag.reshape(2 * N, H),
        dinv_n.reshape(NP), dinv_d.reshape(NP), z2)
    sv = _t2(acc_n, acc_d,
             dinv_n.reshape(NP, 1), dinv_d.reshape(NP, 1),
             net_b.reshape(1, D), dag_b.reshape(1, D),
             action.reshape(1, -1), A1, b1.reshape(1, D), A2,
             b2.reshape(1, D), F1, fb1.reshape(1, D), F2,
             fb2.reshape(1, 1))
    return sv[0, :1]


# speedup vs baseline: 9.4887x; 1.2050x over previous
import jax
import jax.numpy as jnp
from jax import lax
from jax.experimental import pallas as pl
from jax.experimental.pallas import tpu as pltpu
from jax.experimental.pallas import tpu_sc as plsc

N = 10000
D = 128
H = 64
NP = 10240
NN = 2 * NP
NC, NS, L = 2, 16, 16
K = 128
NB = 3968
PE = NB * K
NBT = NB // NS
NBC = NB // (NS * NC)
SL2 = NN // NS
DEG2 = (NN // D, D)


def _scdeg_body(einfo, z1, deg_out,
                eb0, eb1, cb0, cb1, ewb0, ewb1, deg_sp,
                sem_e0, sem_e1, sem_s0, sem_s1):
    c = lax.axis_index("c")
    s = lax.axis_index("s")
    sl = pl.ds(s * SL2, SL2)
    pltpu.sync_copy(z1.at[sl], deg_sp.at[sl])
    plsc.subcore_barrier()

    base = (c * NS + s) * NBC
    pltpu.async_copy(einfo.at[base], eb0, sem_e0)
    pltpu.async_copy(einfo.at[base + 1], eb1, sem_e1)

    bufs = ((eb0, cb0, ewb0, sem_e0, sem_s0),
            (eb1, cb1, ewb1, sem_e1, sem_s1))

    def body(g, _):
        for b, (eb, cb, ewb, sem_e, sem_s) in enumerate(bufs):
            i = g * 2 + b
            pltpu.make_async_copy(einfo.at[0], eb, sem_e).wait()

            @pl.when(g > 0)
            def _():
                pltpu.make_async_copy(z1.at[pl.ds(0, K)], ewb, sem_s).wait()

            for j in range(K // L):
                jj = pl.ds(j * L, L)
                cb[jj] = eb[1, jj]
                ewb[jj] = plsc.bitcast(eb[2, jj], jnp.float32)

            @pl.when(i + 2 < NBC)
            def _():
                pltpu.async_copy(einfo.at[base + i + 2], eb, sem_e)

            pltpu.async_copy(ewb, deg_sp.at[cb], sem_s, add=True)
        return 0

    lax.fori_loop(0, NBC // 2, body, 0)
    pltpu.make_async_copy(z1.at[pl.ds(0, K)], ewb0, sem_s0).wait()
    pltpu.make_async_copy(z1.at[pl.ds(0, K)], ewb1, sem_s1).wait()
    plsc.subcore_barrier()
    pltpu.sync_copy(deg_sp.at[sl], deg_out.at[c, sl])


def _sc_deg(einfo, z1):
    f32 = jnp.float32
    i32 = jnp.int32
    kern = pl.kernel(
        _scdeg_body,
        mesh=plsc.VectorSubcoreMesh(core_axis_name="c", subcore_axis_name="s"),
        compiler_params=pltpu.CompilerParams(
            needs_layout_passes=False, use_tc_tiling_on_sc=False),
        out_type=jax.ShapeDtypeStruct((NC, NN), f32),
        scratch_types=[
            pltpu.VMEM((3, K), i32), pltpu.VMEM((3, K), i32),
            pltpu.VMEM((K,), i32), pltpu.VMEM((K,), i32),
            pltpu.VMEM((K,), f32), pltpu.VMEM((K,), f32),
            pltpu.VMEM_SHARED((NN,), f32),
            pltpu.SemaphoreType.DMA, pltpu.SemaphoreType.DMA,
            pltpu.SemaphoreType.DMA, pltpu.SemaphoreType.DMA,
        ],
    )
    return kern(einfo, z1)


def _mm_body(nf, df, nw, dw, dg, xw, dv):
    xw[0] = jnp.dot(nf[...], nw[...], preferred_element_type=jnp.float32)
    xw[1] = jnp.dot(df[...], dw[...], preferred_element_type=jnp.float32)

    @pl.when(pl.program_id(0) == 0)
    def _():
        degsum = dg[0] + dg[1]
        dv[...] = jnp.where(degsum > 0, lax.rsqrt(degsum), 0.0)


def _t1(net_feat, net_W, dag_feat, dag_W, deg):
    MB = 640
    cs = lambda shape: pl.BlockSpec(shape, lambda m: tuple(0 for _ in shape))
    return pl.pallas_call(
        _mm_body,
        grid=(NP // MB,),
        in_specs=[
            pl.BlockSpec((MB, D), lambda m: (m, 0)),
            pl.BlockSpec((MB, D), lambda m: (m, 0)),
            cs((D, D)), cs((D, D)),
            cs((NC,) + DEG2),
        ],
        out_specs=[
            pl.BlockSpec((2, MB, D), lambda m: (0, m, 0)),
            cs(DEG2),
        ],
        out_shape=[
            jax.ShapeDtypeStruct((2, NP, D), jnp.float32),
            jax.ShapeDtypeStruct(DEG2, jnp.float32),
        ],
    )(net_feat, dag_feat, net_W, dag_W, deg.reshape((NC,) + DEG2))


def _scmsg_body(einfo, xwi, dinv, z2, acc_out,
                eb0, eb1, ib0, ib1, cb0, cb1, rows0, rows1, nbuf, dinv_full,
                acc_sp, sem_e0, sem_e1, sem_g0, sem_g1, sem_s0, sem_s1):
    c = lax.axis_index("c")
    s = lax.axis_index("s")
    sl = pl.ds(s * SL2, SL2)
    pltpu.sync_copy(z2.at[sl], acc_sp.at[sl])
    pltpu.sync_copy(dinv, dinv_full)

    base = s * NBT
    pltpu.async_copy(einfo.at[base], eb0, sem_e0)
    pltpu.async_copy(einfo.at[base + 1], eb1, sem_e1)
    plsc.subcore_barrier()

    bufs = ((eb0, ib0, cb0, rows0, sem_e0, sem_g0, sem_s0),
            (eb1, ib1, cb1, rows1, sem_e1, sem_g1, sem_s1))

    def body(g, _):
        for b, (eb, ib, cb, rows, sem_e, sem_g, sem_s) in enumerate(bufs):
            i = g * 2 + b
            pltpu.make_async_copy(einfo.at[0], eb, sem_e).wait()
            for j in range(K // L):
                jj = pl.ds(j * L, L)
                ib[jj] = eb[0, jj] * 2 + c
                cb[jj] = eb[1, jj]

            @pl.when(g > 0)
            def _():
                pltpu.make_async_copy(xwi.at[pl.ds(0, K)], rows, sem_s).wait()

            pltpu.async_copy(xwi.at[ib], rows, sem_g)
            for j in range(K // L):
                jj = pl.ds(j * L, L)
                r16 = eb[0, jj]
                ew16 = plsc.bitcast(eb[2, jj], jnp.float32)
                nbuf[jj] = ew16 * plsc.load_gather(dinv_full, [r16])

            @pl.when(i + 2 < NBT)
            def _():
                pltpu.async_copy(einfo.at[base + i + 2], eb, sem_e)

            pltpu.make_async_copy(xwi.at[pl.ds(0, K)], rows, sem_g).wait()
            for j in range(K // L):
                for e in range(L):
                    r = j * L + e
                    ns = plsc.load_gather(
                        nbuf, [jnp.full((L,), r, jnp.int32)])
                    for cc in range(H // L):
                        rows[r, cc * L:(cc + 1) * L] = (
                            rows[r, cc * L:(cc + 1) * L] * ns)
            pltpu.async_copy(rows, acc_sp.at[cb], sem_s, add=True)
        return 0

    lax.fori_loop(0, NBT // 2, body, 0)
    pltpu.make_async_copy(xwi.at[pl.ds(0, K)], rows0, sem_s0).wait()
    pltpu.make_async_copy(xwi.at[pl.ds(0, K)], rows1, sem_s1).wait()
    plsc.subcore_barrier()

    pltpu.sync_copy(acc_sp.at[sl], acc_out.at[c, sl])


def _sc_msg(einfo, xwi, dinv, z2):
    f32 = jnp.float32
    i32 = jnp.int32
    kern = pl.kernel(
        _scmsg_body,
        mesh=plsc.VectorSubcoreMesh(core_axis_name="c", subcore_axis_name="s"),
        compiler_params=pltpu.CompilerParams(
            needs_layout_passes=False, use_tc_tiling_on_sc=False),
        out_type=jax.ShapeDtypeStruct((NC, NN, H), f32),
        scratch_types=[
            pltpu.VMEM((3, K), i32), pltpu.VMEM((3, K), i32),
            pltpu.VMEM((K,), i32), pltpu.VMEM((K,), i32),
            pltpu.VMEM((K,), i32), pltpu.VMEM((K,), i32),
            pltpu.VMEM((K, H), f32), pltpu.VMEM((K, H), f32),
            pltpu.VMEM((K,), f32),
            pltpu.VMEM((NN,), f32),
            pltpu.VMEM_SHARED((NN, H), f32),
            pltpu.SemaphoreType.DMA, pltpu.SemaphoreType.DMA,
            pltpu.SemaphoreType.DMA, pltpu.SemaphoreType.DMA,
            pltpu.SemaphoreType.DMA, pltpu.SemaphoreType.DMA,
        ],
    )
    return kern(einfo, xwi, dinv, z2)


def _t2_body(acc, dinv, bsel, act, A1, b1, A2, b2, F1, fb1, F2, fb2,
             out, s_ref):
    m = pl.program_id(0)
    nblk = pl.num_programs(0)
    BLK = acc.shape[1]

    @pl.when(m == 0)
    def _():
        s_ref[...] = jnp.zeros_like(s_ref)

    r = m * BLK + lax.broadcasted_iota(jnp.int32, (BLK, 1), 0)
    mask = ((r < N) | ((r >= NP) & (r < NP + N))).astype(jnp.float32)
    g = m // (nblk // 2)

    dv = dinv[...]
    bg = bsel[pl.ds(g, 1), :]
    v0 = jax.nn.relu(dv * acc[0] + bg[0:1, 0:H])
    v1 = jax.nn.relu(dv * acc[1] + bg[0:1, H:D])
    s0 = jnp.sum(v0 * mask, axis=0, keepdims=True)
    s1 = jnp.sum(v1 * mask, axis=0, keepdims=True)
    s_ref[pl.ds(g, 1), 0:H] += s0
    s_ref[pl.ds(g, 1), H:D] += s1

    @pl.when(m == nblk - 1)
    def _():
        inv_n = jnp.float32(1.0 / N)
        emb_n = s_ref[0:1, :] * inv_n
        emb_d = s_ref[1:2, :] * inv_n
        hh = act[...] @ A1[...] + b1[...]
        hh = hh * jnp.tanh(jax.nn.softplus(hh))
        ae = hh @ A2[...] + b2[...]
        h2 = jax.nn.relu(
            emb_n @ F1[0:D, :] + emb_d @ F1[D:2 * D, :]
            + ae @ F1[2 * D:3 * D, :] + fb1[...])
        sv = jnp.sum(h2 * F2[...].T, axis=1, keepdims=True) + fb2[...]
        out[...] = jnp.broadcast_to(sv, out.shape)


def _t2(acc, dinv, bsel, act, A1, b1, A2, b2, F1, fb1, F2, fb2):
    BLK = 512
    nblk = NN // BLK
    cs = lambda shape: pl.BlockSpec(shape, lambda m: tuple(0 for _ in shape))
    return pl.pallas_call(
        _t2_body,
        grid=(nblk,),
        in_specs=[
            pl.BlockSpec((NC, BLK, H), lambda m: (0, m, 0)),
            pl.BlockSpec((BLK, 1), lambda m: (m, 0)),
            cs((2, D)),
            cs((1, 512)),
            cs((512, D)), cs((1, D)),
            cs((D, D)), cs((1, D)),
            cs((3 * D, D)), cs((1, D)),
            cs((D, 1)), cs((1, 1)),
        ],
        out_specs=pl.BlockSpec((1, D), lambda m: (0, 0)),
        out_shape=jax.ShapeDtypeStruct((1, D), jnp.float32),
        scratch_shapes=[pltpu.VMEM((8, D), jnp.float32)],
    )(acc, dinv, bsel, act, A1, b1, A2, b2, F1, fb1, F2, fb2)


def _prep(nei, new, dei, dew):
    i32 = jnp.int32
    f32 = jnp.float32
    ar = jnp.arange(N, dtype=i32)
    pad = PE - (new.shape[0] + dew.shape[0] + 2 * N)
    row = jnp.concatenate(
        [nei[0], ar, NP + dei[0], NP + ar, jnp.zeros((pad,), i32)])
    col = jnp.concatenate(
        [nei[1], ar, NP + dei[1], NP + ar, jnp.full((pad,), N, i32)])
    ew = jnp.concatenate(
        [new, jnp.ones((N,), f32), dew, jnp.ones((N,), f32),
         jnp.zeros((pad,), f32)])
    einfo = jnp.stack([row, col, lax.bitcast_convert_type(ew, i32)])
    return einfo.reshape(3, NB, K).transpose(1, 0, 2)


def kernel(net_feat, net_edge_index, net_edge_weights, dag_feat,
           dag_edge_index, dag_edge_weights, action, net_W, net_b, dag_W,
           dag_b, A1, b1, A2, b2, F1, fb1, F2, fb2):
    einfo = _prep(net_edge_index, net_edge_weights,
                  dag_edge_index, dag_edge_weights)
    z1 = jnp.zeros((NN,), jnp.float32)
    z2 = jnp.zeros((NN, H), jnp.float32)
    deg = _sc_deg(einfo, z1)
    pad_rows = jnp.zeros((NP - N, D), jnp.float32)
    nf = jnp.concatenate([net_feat, pad_rows])
    df = jnp.concatenate([dag_feat, pad_rows])
    xw, dinv = _t1(nf, net_W, df, dag_W, deg)
    acc = _sc_msg(einfo, xw.reshape(4 * NP, H), dinv.reshape(NN), z2)
    bsel = jnp.stack([net_b, dag_b])
    sv = _t2(acc, dinv.reshape(NN, 1), bsel, action.reshape(1, -1),
             A1, b1.reshape(1, D), A2, b2.reshape(1, D),
             F1, fb1.reshape(1, D), F2, fb2.reshape(1, 1))
    return sv[0, :1]
